# Initial kernel scaffold; baseline (speedup 1.0000x reference)
import jax
import jax.numpy as jnp
from jax.experimental import pallas as pl


def kernel(state, action, edge_index, edge_attr, batch, W_in, b_in, W_msg0, W_self0, b0, W_msg1, W_self1, b1, W_out, b_out):
    raise NotImplementedError("write your pallas kernel here")


# speedup vs baseline: 5.7022x; 5.7022x over previous
import functools
import jax
import jax.numpy as jnp
from jax import lax
from jax.experimental import pallas as pl
from jax.experimental.pallas import tpu as pltpu
from jax.experimental.pallas import tpu_sc as plsc

N = 100000
E = 1600000
C = 32
G = 64

W_E = 1024
E_PAD = 1605632
NWIN_B = 98
NWIN_A = 49
NACC = 102400
ZCH = 6400


def _sc_mesh():
    return plsc.VectorSubcoreMesh(core_axis_name="c", subcore_axis_name="s")


def _sedeg_body(ea8_hbm, dst_hbm, z8_hbm, out_hbm, acc8, ea_v, idx_v, sem):
    c = lax.axis_index("c")
    s = lax.axis_index("s")
    pltpu.sync_copy(z8_hbm, acc8.at[pl.ds(s * ZCH, ZCH)])
    plsc.subcore_barrier()
    t = c * 16 + s
    base_edge = t * (NWIN_A * W_E)

    def body(w, _):
        e0 = base_edge + w * W_E
        pltpu.sync_copy(dst_hbm.at[pl.ds(e0, W_E)], idx_v)
        pltpu.sync_copy(ea8_hbm.at[pl.ds(e0, W_E)], ea_v)
        pltpu.sync_copy(ea_v, acc8.at[idx_v], add=True)
        return 0

    lax.fori_loop(0, NWIN_A, body, 0)
    plsc.subcore_barrier()
    pltpu.sync_copy(acc8.at[pl.ds(s * ZCH, ZCH)], out_hbm.at[c, pl.ds(s * ZCH, ZCH)])


def _make_sedeg_kernel():
    return pl.kernel(
        _sedeg_body,
        out_type=jax.ShapeDtypeStruct((2, NACC, 8), jnp.float32),
        mesh=_sc_mesh(),
        compiler_params=pltpu.CompilerParams(use_tc_tiling_on_sc=False),
        scratch_types=[
            pltpu.VMEM_SHARED((NACC, 8), jnp.float32),
            pltpu.VMEM((W_E, 8), jnp.float32),
            pltpu.VMEM((W_E,), jnp.int32),
            pltpu.SemaphoreType.DMA,
        ],
    )


def _spmm_body(hl_hbm, hr_hbm, src_hbm, dst_hbm, z16_hbm, out_hbm,
               acc, rows_v, sidx_v, didx_v, sem):
    c = lax.axis_index("c")
    s = lax.axis_index("s")
    pltpu.sync_copy(z16_hbm, acc.at[pl.ds(s * ZCH, ZCH)])
    plsc.subcore_barrier()
    base_edge = s * (NWIN_B * W_E)

    def body(w, _):
        e0 = base_edge + w * W_E
        pltpu.sync_copy(src_hbm.at[pl.ds(e0, W_E)], sidx_v)
        pltpu.sync_copy(dst_hbm.at[pl.ds(e0, W_E)], didx_v)

        @pl.when(c == 0)
        def _():
            pltpu.async_copy(hl_hbm.at[sidx_v], rows_v, sem).wait()

        @pl.when(c == 1)
        def _():
            pltpu.async_copy(hr_hbm.at[sidx_v], rows_v, sem).wait()

        pltpu.sync_copy(rows_v, acc.at[didx_v], add=True)
        return 0

    lax.fori_loop(0, NWIN_B, body, 0)
    plsc.subcore_barrier()
    pltpu.sync_copy(acc.at[pl.ds(s * ZCH, ZCH)], out_hbm.at[c, pl.ds(s * ZCH, ZCH)])


def _make_spmm_kernel():
    return pl.kernel(
        _spmm_body,
        out_type=jax.ShapeDtypeStruct((2, NACC, 16), jnp.float32),
        mesh=_sc_mesh(),
        compiler_params=pltpu.CompilerParams(use_tc_tiling_on_sc=False),
        scratch_types=[
            pltpu.VMEM_SHARED((NACC, 16), jnp.float32),
            pltpu.VMEM((W_E, 16), jnp.float32),
            pltpu.VMEM((W_E,), jnp.int32),
            pltpu.VMEM((W_E,), jnp.int32),
            pltpu.SemaphoreType.DMA,
        ],
    )


BLK = 800
NBLK = N // BLK


def _readin_body(state_ref, action_ref, wins_ref, wina_ref, b_ref, hl_ref, hr_ref):
    h = state_ref[...] @ wins_ref[...] + action_ref[...] @ wina_ref[...] + b_ref[...]
    h = jnp.where(h > 0, h, 0.01 * h)
    hl_ref[...] = h[:, :16]
    hr_ref[...] = h[:, 16:]


def _readin(state, action, w_s, w_a, b):
    return pl.pallas_call(
        _readin_body,
        grid=(NBLK,),
        in_specs=[
            pl.BlockSpec((BLK, 96), lambda i: (i, 0)),
            pl.BlockSpec((BLK, 32), lambda i: (i, 0)),
            pl.BlockSpec((96, 32), lambda i: (0, 0)),
            pl.BlockSpec((32, 32), lambda i: (0, 0)),
            pl.BlockSpec((1, 32), lambda i: (0, 0)),
        ],
        out_specs=[
            pl.BlockSpec((BLK, 16), lambda i: (i, 0)),
            pl.BlockSpec((BLK, 16), lambda i: (i, 0)),
        ],
        out_shape=[
            jax.ShapeDtypeStruct((N, 16), jnp.float32),
            jax.ShapeDtypeStruct((N, 16), jnp.float32),
        ],
    )(state, action, w_s, w_a, b)


def _layer_body(hl_ref, hr_ref, tl_ref, tr_ref, sd0_ref, sd1_ref,
                wself_ref, b_ref, a_ref, b4_ref, ol_ref, or_ref):
    h = jnp.concatenate([hl_ref[...], hr_ref[...]], axis=1)
    t = jnp.concatenate([tl_ref[...], tr_ref[...]], axis=1)
    sd = sd0_ref[...] + sd1_ref[...]
    se = sd[:, :4]
    deg = jnp.clip(sd[:, 4:5], 1.0, None)
    agg = (t @ a_ref[...] + se @ b4_ref[...]) / deg
    hn = h @ wself_ref[...] + b_ref[...] + agg
    hn = jnp.where(hn > 0, hn, 0.01 * hn)
    ol_ref[...] = hn[:, :16]
    or_ref[...] = hn[:, 16:]


def _layer(hl, hr, tl, tr, sd0, sd1, wself, b, a, b4):
    return pl.pallas_call(
        _layer_body,
        grid=(NBLK,),
        in_specs=[
            pl.BlockSpec((BLK, 16), lambda i: (i, 0)),
            pl.BlockSpec((BLK, 16), lambda i: (i, 0)),
            pl.BlockSpec((BLK, 16), lambda i: (i, 0)),
            pl.BlockSpec((BLK, 16), lambda i: (i, 0)),
            pl.BlockSpec((BLK, 8), lambda i: (i, 0)),
            pl.BlockSpec((BLK, 8), lambda i: (i, 0)),
            pl.BlockSpec((32, 32), lambda i: (0, 0)),
            pl.BlockSpec((1, 32), lambda i: (0, 0)),
            pl.BlockSpec((32, 32), lambda i: (0, 0)),
            pl.BlockSpec((4, 32), lambda i: (0, 0)),
        ],
        out_specs=[
            pl.BlockSpec((BLK, 16), lambda i: (i, 0)),
            pl.BlockSpec((BLK, 16), lambda i: (i, 0)),
        ],
        out_shape=[
            jax.ShapeDtypeStruct((N, 16), jnp.float32),
            jax.ShapeDtypeStruct((N, 16), jnp.float32),
        ],
    )(hl, hr, tl, tr, sd0, sd1, wself, b, a, b4)


def _readout_body(hl_ref, hr_ref, batch_ref, wout_ref, bout_ref, out_ref,
                  sums_ref, cnt_ref):
    i = pl.program_id(0)

    @pl.when(i == 0)
    def _():
        sums_ref[...] = jnp.zeros_like(sums_ref)
        cnt_ref[...] = jnp.zeros_like(cnt_ref)

    h = jnp.concatenate([hl_ref[...], hr_ref[...]], axis=1)
    y = h @ wout_ref[...] + bout_ref[...]
    gids = jax.lax.broadcasted_iota(jnp.int32, (1, G), 1)
    onehot = (batch_ref[...] == gids).astype(jnp.float32)
    sums_ref[...] += jnp.sum(onehot * y, axis=0, keepdims=True)
    cnt_ref[...] += jnp.sum(onehot, axis=0, keepdims=True)

    @pl.when(i == NBLK - 1)
    def _():
        out_ref[...] = sums_ref[...] / jnp.clip(cnt_ref[...], 1.0, None)


def _readout(hl, hr, batch_col, wout, bout):
    return pl.pallas_call(
        _readout_body,
        grid=(NBLK,),
        in_specs=[
            pl.BlockSpec((BLK, 16), lambda i: (i, 0)),
            pl.BlockSpec((BLK, 16), lambda i: (i, 0)),
            pl.BlockSpec((BLK, 1), lambda i: (i, 0)),
            pl.BlockSpec((32, 1), lambda i: (0, 0)),
            pl.BlockSpec((1, 1), lambda i: (0, 0)),
        ],
        out_specs=pl.BlockSpec((1, G), lambda i: (0, 0)),
        out_shape=jax.ShapeDtypeStruct((1, G), jnp.float32),
        scratch_shapes=[
            pltpu.VMEM((1, G), jnp.float32),
            pltpu.VMEM((1, G), jnp.float32),
        ],
    )(hl, hr, batch_col, wout, bout)


@jax.jit
def kernel(state, action, edge_index, edge_attr, batch,
           W_in, b_in, W_msg0, W_self0, b0, W_msg1, W_self1, b1, W_out, b_out):
    src = edge_index[0]
    dst = edge_index[1]
    pad = E_PAD - E
    pad_i = lax.iota(jnp.int32, pad)
    src_p = jnp.concatenate([src, pad_i % 2048])
    dst_p = jnp.concatenate([dst, N + (pad_i % 2048)])

    ea8 = jnp.concatenate(
        [edge_attr,
         jnp.ones((E, 1), jnp.float32),
         jnp.zeros((E, 3), jnp.float32)], axis=1)
    ea8 = jnp.concatenate([ea8, jnp.zeros((pad, 8), jnp.float32)], axis=0)

    z8 = jnp.zeros((ZCH, 8), jnp.float32)
    z16 = jnp.zeros((ZCH, 16), jnp.float32)

    sedeg = _make_sedeg_kernel()(ea8, dst_p, z8)
    sd0 = sedeg[0, :N]
    sd1 = sedeg[1, :N]

    hl, hr = _readin(state, action, W_in[:96], W_in[96:], b_in.reshape(1, C))

    spmm = _make_spmm_kernel()
    t = spmm(hl, hr, src_p, dst_p, z16)
    hl, hr = _layer(hl, hr, t[0, :N], t[1, :N], sd0, sd1,
                    W_self0, b0.reshape(1, C), W_msg0[:C], W_msg0[C:])

    t = spmm(hl, hr, src_p, dst_p, z16)
    hl, hr = _layer(hl, hr, t[0, :N], t[1, :N], sd0, sd1,
                    W_self1, b1.reshape(1, C), W_msg1[:C], W_msg1[C:])

    batch_col = batch.reshape(N, 1)
    out = _readout(hl, hr, batch_col, W_out, b_out.reshape(1, 1))
    return out.reshape(G, 1)


# speedup vs baseline: 5.9642x; 1.0459x over previous
import functools
import jax
import jax.numpy as jnp
from jax import lax
from jax.experimental import pallas as pl
from jax.experimental.pallas import tpu as pltpu
from jax.experimental.pallas import tpu_sc as plsc

N = 100000
E = 1600000
C = 32
G = 64

W_E = 768
E_PAD = 1622016
NWIN_B = 132
NWIN_A = 66
NACC = 102400
ZCH = 6400


def _sc_mesh():
    return plsc.VectorSubcoreMesh(core_axis_name="c", subcore_axis_name="s")


def _sedeg_body(ea8_hbm, dst_hbm, z8_hbm, out_hbm, acc8, ea_v, idx_v, lsem, ssem):
    c = lax.axis_index("c")
    s = lax.axis_index("s")
    pltpu.sync_copy(z8_hbm, acc8.at[pl.ds(s * ZCH, ZCH)])
    plsc.subcore_barrier()
    t = c * 16 + s
    base_edge = t * (NWIN_A * W_E)

    def load(w, b):
        e0 = base_edge + w * W_E
        pltpu.sync_copy(dst_hbm.at[pl.ds(e0, W_E)], idx_v.at[b])
        pltpu.async_copy(ea8_hbm.at[pl.ds(e0, W_E)], ea_v.at[b], lsem)

    load(0, 0)

    def body(w, _):
        b = lax.rem(w, 2)
        nb = 1 - b

        @pl.when(w >= 1)
        def _():
            pltpu.make_async_copy(ea_v.at[nb], acc8.at[idx_v.at[nb]], ssem).wait()

        @pl.when(w + 1 < NWIN_A)
        def _():
            load(w + 1, nb)

        pltpu.make_async_copy(ea8_hbm.at[pl.ds(0, W_E)], ea_v.at[b], lsem).wait()
        pltpu.async_copy(ea_v.at[b], acc8.at[idx_v.at[b]], ssem, add=True)
        return 0

    lax.fori_loop(0, NWIN_A, body, 0)
    lb = (NWIN_A - 1) % 2
    pltpu.make_async_copy(ea_v.at[lb], acc8.at[idx_v.at[lb]], ssem).wait()
    plsc.subcore_barrier()
    pltpu.sync_copy(acc8.at[pl.ds(s * ZCH, ZCH)], out_hbm.at[c, pl.ds(s * ZCH, ZCH)])


def _make_sedeg_kernel():
    return pl.kernel(
        _sedeg_body,
        out_type=jax.ShapeDtypeStruct((2, NACC, 8), jnp.float32),
        mesh=_sc_mesh(),
        compiler_params=pltpu.CompilerParams(use_tc_tiling_on_sc=False),
        scratch_types=[
            pltpu.VMEM_SHARED((NACC, 8), jnp.float32),
            pltpu.VMEM((2, W_E, 8), jnp.float32),
            pltpu.VMEM((2, W_E), jnp.int32),
            pltpu.SemaphoreType.DMA,
            pltpu.SemaphoreType.DMA,
        ],
    )


def _spmm_body(hl_hbm, hr_hbm, src_hbm, dst_hbm, z16_hbm, out_hbm,
               acc, rows_v, sidx_v, didx_v, gsem, ssem):
    c = lax.axis_index("c")
    s = lax.axis_index("s")
    pltpu.sync_copy(z16_hbm, acc.at[pl.ds(s * ZCH, ZCH)])
    plsc.subcore_barrier()
    base_edge = s * (NWIN_B * W_E)

    def load(w, b):
        e0 = base_edge + w * W_E
        pltpu.sync_copy(src_hbm.at[pl.ds(e0, W_E)], sidx_v.at[b])
        pltpu.sync_copy(dst_hbm.at[pl.ds(e0, W_E)], didx_v.at[b])

        @pl.when(c == 0)
        def _():
            pltpu.async_copy(hl_hbm.at[sidx_v.at[b]], rows_v.at[b], gsem)

        @pl.when(c == 1)
        def _():
            pltpu.async_copy(hr_hbm.at[sidx_v.at[b]], rows_v.at[b], gsem)

    load(0, 0)

    def body(w, _):
        b = lax.rem(w, 2)
        nb = 1 - b

        @pl.when(w >= 1)
        def _():
            pltpu.make_async_copy(rows_v.at[nb], acc.at[didx_v.at[nb]], ssem).wait()

        @pl.when(w + 1 < NWIN_B)
        def _():
            load(w + 1, nb)

        pltpu.make_async_copy(hl_hbm.at[sidx_v.at[b]], rows_v.at[b], gsem).wait()
        pltpu.async_copy(rows_v.at[b], acc.at[didx_v.at[b]], ssem, add=True)
        return 0

    lax.fori_loop(0, NWIN_B, body, 0)
    lb = (NWIN_B - 1) % 2
    pltpu.make_async_copy(rows_v.at[lb], acc.at[didx_v.at[lb]], ssem).wait()
    plsc.subcore_barrier()
    pltpu.sync_copy(acc.at[pl.ds(s * ZCH, ZCH)], out_hbm.at[c, pl.ds(s * ZCH, ZCH)])


def _make_spmm_kernel():
    return pl.kernel(
        _spmm_body,
        out_type=jax.ShapeDtypeStruct((2, NACC, 16), jnp.float32),
        mesh=_sc_mesh(),
        compiler_params=pltpu.CompilerParams(use_tc_tiling_on_sc=False),
        scratch_types=[
            pltpu.VMEM_SHARED((NACC, 16), jnp.float32),
            pltpu.VMEM((2, W_E, 16), jnp.float32),
            pltpu.VMEM((2, W_E), jnp.int32),
            pltpu.VMEM((2, W_E), jnp.int32),
            pltpu.SemaphoreType.DMA,
            pltpu.SemaphoreType.DMA,
        ],
    )


BLK = 800
NBLK = N // BLK


def _readin_body(state_ref, action_ref, wins_ref, wina_ref, b_ref, hl_ref, hr_ref):
    h = state_ref[...] @ wins_ref[...] + action_ref[...] @ wina_ref[...] + b_ref[...]
    h = jnp.where(h > 0, h, 0.01 * h)
    hl_ref[...] = h[:, :16]
    hr_ref[...] = h[:, 16:]


def _readin(state, action, w_s, w_a, b):
    return pl.pallas_call(
        _readin_body,
        grid=(NBLK,),
        in_specs=[
            pl.BlockSpec((BLK, 96), lambda i: (i, 0)),
            pl.BlockSpec((BLK, 32), lambda i: (i, 0)),
            pl.BlockSpec((96, 32), lambda i: (0, 0)),
            pl.BlockSpec((32, 32), lambda i: (0, 0)),
            pl.BlockSpec((1, 32), lambda i: (0, 0)),
        ],
        out_specs=[
            pl.BlockSpec((BLK, 16), lambda i: (i, 0)),
            pl.BlockSpec((BLK, 16), lambda i: (i, 0)),
        ],
        out_shape=[
            jax.ShapeDtypeStruct((N, 16), jnp.float32),
            jax.ShapeDtypeStruct((N, 16), jnp.float32),
        ],
    )(state, action, w_s, w_a, b)


def _layer_body(hl_ref, hr_ref, tl_ref, tr_ref, sd0_ref, sd1_ref,
                wself_ref, b_ref, a_ref, b4_ref, ol_ref, or_ref):
    h = jnp.concatenate([hl_ref[...], hr_ref[...]], axis=1)
    t = jnp.concatenate([tl_ref[...], tr_ref[...]], axis=1)
    sd = sd0_ref[...] + sd1_ref[...]
    se = sd[:, :4]
    deg = jnp.clip(sd[:, 4:5], 1.0, None)
    agg = (t @ a_ref[...] + se @ b4_ref[...]) / deg
    hn = h @ wself_ref[...] + b_ref[...] + agg
    hn = jnp.where(hn > 0, hn, 0.01 * hn)
    ol_ref[...] = hn[:, :16]
    or_ref[...] = hn[:, 16:]


def _layer(hl, hr, tl, tr, sd0, sd1, wself, b, a, b4):
    return pl.pallas_call(
        _layer_body,
        grid=(NBLK,),
        in_specs=[
            pl.BlockSpec((BLK, 16), lambda i: (i, 0)),
            pl.BlockSpec((BLK, 16), lambda i: (i, 0)),
            pl.BlockSpec((BLK, 16), lambda i: (i, 0)),
            pl.BlockSpec((BLK, 16), lambda i: (i, 0)),
            pl.BlockSpec((BLK, 8), lambda i: (i, 0)),
            pl.BlockSpec((BLK, 8), lambda i: (i, 0)),
            pl.BlockSpec((32, 32), lambda i: (0, 0)),
            pl.BlockSpec((1, 32), lambda i: (0, 0)),
            pl.BlockSpec((32, 32), lambda i: (0, 0)),
            pl.BlockSpec((4, 32), lambda i: (0, 0)),
        ],
        out_specs=[
            pl.BlockSpec((BLK, 16), lambda i: (i, 0)),
            pl.BlockSpec((BLK, 16), lambda i: (i, 0)),
        ],
        out_shape=[
            jax.ShapeDtypeStruct((N, 16), jnp.float32),
            jax.ShapeDtypeStruct((N, 16), jnp.float32),
        ],
    )(hl, hr, tl, tr, sd0, sd1, wself, b, a, b4)


def _readout_body(hl_ref, hr_ref, batch_ref, wout_ref, bout_ref, out_ref,
                  sums_ref, cnt_ref):
    i = pl.program_id(0)

    @pl.when(i == 0)
    def _():
        sums_ref[...] = jnp.zeros_like(sums_ref)
        cnt_ref[...] = jnp.zeros_like(cnt_ref)

    h = jnp.concatenate([hl_ref[...], hr_ref[...]], axis=1)
    y = h @ wout_ref[...] + bout_ref[...]
    gids = jax.lax.broadcasted_iota(jnp.int32, (1, G), 1)
    onehot = (batch_ref[...] == gids).astype(jnp.float32)
    sums_ref[...] += jnp.sum(onehot * y, axis=0, keepdims=True)
    cnt_ref[...] += jnp.sum(onehot, axis=0, keepdims=True)

    @pl.when(i == NBLK - 1)
    def _():
        out_ref[...] = sums_ref[...] / jnp.clip(cnt_ref[...], 1.0, None)


def _readout(hl, hr, batch_col, wout, bout):
    return pl.pallas_call(
        _readout_body,
        grid=(NBLK,),
        in_specs=[
            pl.BlockSpec((BLK, 16), lambda i: (i, 0)),
            pl.BlockSpec((BLK, 16), lambda i: (i, 0)),
            pl.BlockSpec((BLK, 1), lambda i: (i, 0)),
            pl.BlockSpec((32, 1), lambda i: (0, 0)),
            pl.BlockSpec((1, 1), lambda i: (0, 0)),
        ],
        out_specs=pl.BlockSpec((1, G), lambda i: (0, 0)),
        out_shape=jax.ShapeDtypeStruct((1, G), jnp.float32),
        scratch_shapes=[
            pltpu.VMEM((1, G), jnp.float32),
            pltpu.VMEM((1, G), jnp.float32),
        ],
    )(hl, hr, batch_col, wout, bout)


@jax.jit
def kernel(state, action, edge_index, edge_attr, batch,
           W_in, b_in, W_msg0, W_self0, b0, W_msg1, W_self1, b1, W_out, b_out):
    src = edge_index[0]
    dst = edge_index[1]
    pad = E_PAD - E
    pad_i = lax.iota(jnp.int32, pad)
    src_p = jnp.concatenate([src, pad_i % 2048])
    dst_p = jnp.concatenate([dst, N + (pad_i % 2048)])

    ea8 = jnp.concatenate(
        [edge_attr,
         jnp.ones((E, 1), jnp.float32),
         jnp.zeros((E, 3), jnp.float32)], axis=1)
    ea8 = jnp.concatenate([ea8, jnp.zeros((pad, 8), jnp.float32)], axis=0)

    z8 = jnp.zeros((ZCH, 8), jnp.float32)
    z16 = jnp.zeros((ZCH, 16), jnp.float32)

    sedeg = _make_sedeg_kernel()(ea8, dst_p, z8)
    sd0 = sedeg[0, :N]
    sd1 = sedeg[1, :N]

    hl, hr = _readin(state, action, W_in[:96], W_in[96:], b_in.reshape(1, C))

    spmm = _make_spmm_kernel()
    t = spmm(hl, hr, src_p, dst_p, z16)
    hl, hr = _layer(hl, hr, t[0, :N], t[1, :N], sd0, sd1,
                    W_self0, b0.reshape(1, C), W_msg0[:C], W_msg0[C:])

    t = spmm(hl, hr, src_p, dst_p, z16)
    hl, hr = _layer(hl, hr, t[0, :N], t[1, :N], sd0, sd1,
                    W_self1, b1.reshape(1, C), W_msg1[:C], W_msg1[C:])

    batch_col = batch.reshape(N, 1)
    out = _readout(hl, hr, batch_col, W_out, b_out.reshape(1, 1))
    return out.reshape(G, 1)


# speedup vs baseline: 7.4132x; 1.2430x over previous
import functools
import jax
import jax.numpy as jnp
from jax import lax
from jax.experimental import pallas as pl
from jax.experimental.pallas import tpu as pltpu
from jax.experimental.pallas import tpu_sc as plsc

N = 100000
E = 1600000
C = 32
G = 64

ZCH = N // 16

W_A = 1000
NWIN_A = 50
CKW_A = 10
NCH_A = NWIN_A // CKW_A
W_B = 800
NWIN_B = 125


def _sc_mesh():
    return plsc.VectorSubcoreMesh(core_axis_name="c", subcore_axis_name="s")


def _sedeg_body(ea8_hbm, dst_hbm, z8_hbm, out_hbm,
                acc8, ea_v, idx_v, lsem, ssem, isem):
    c = lax.axis_index("c")
    s = lax.axis_index("s")
    pltpu.sync_copy(z8_hbm, acc8.at[pl.ds(s * ZCH, ZCH)])
    t = c * 16 + s
    base_edge = t * (NWIN_A * W_A)
    plsc.subcore_barrier()

    def load_chunk(k):
        pltpu.async_copy(
            dst_hbm.at[pl.ds(base_edge + k * (CKW_A * W_A), CKW_A * W_A)],
            idx_v.at[lax.rem(k, 2)], isem)

    pltpu.sync_copy(dst_hbm.at[pl.ds(base_edge, CKW_A * W_A)], idx_v.at[0])
    load_chunk(1)

    def load(w, b):
        pltpu.async_copy(ea8_hbm.at[pl.ds(base_edge + w * W_A, W_A)],
                         ea_v.at[b], lsem)

    load(0, 0)
    dummy_idx = idx_v.at[0].at[pl.ds(0, W_A)]

    def body(w, _):
        b = lax.rem(w, 2)
        nb = 1 - b
        j = lax.rem(w, CKW_A)
        k = lax.div(w, CKW_A)
        idx_w = idx_v.at[lax.rem(k, 2)].at[pl.ds(j * W_A, W_A)]

        @pl.when(w >= 1)
        def _():
            pltpu.make_async_copy(ea_v.at[nb], acc8.at[dummy_idx],
                                  ssem).wait()

        @pl.when((j == 0) & (k >= 1))
        def _():
            pltpu.make_async_copy(dst_hbm.at[pl.ds(0, CKW_A * W_A)],
                                  idx_v.at[lax.rem(k, 2)], isem).wait()

        @pl.when((j == 0) & (k >= 1) & (k + 1 < NCH_A))
        def _():
            load_chunk(k + 1)

        @pl.when(w + 1 < NWIN_A)
        def _():
            load(w + 1, nb)

        pltpu.make_async_copy(ea8_hbm.at[pl.ds(0, W_A)], ea_v.at[b], lsem).wait()
        pltpu.async_copy(ea_v.at[b], acc8.at[idx_w], ssem, add=True)
        return 0

    lax.fori_loop(0, NWIN_A, body, 0)
    lb = (NWIN_A - 1) % 2
    pltpu.make_async_copy(ea_v.at[lb], acc8.at[dummy_idx], ssem).wait()
    plsc.subcore_barrier()
    pltpu.sync_copy(acc8.at[pl.ds(s * ZCH, ZCH)],
                    out_hbm.at[c, pl.ds(s * ZCH, ZCH)])


def _make_sedeg_kernel():
    return pl.kernel(
        _sedeg_body,
        out_type=jax.ShapeDtypeStruct((2, N, 8), jnp.float32),
        mesh=_sc_mesh(),
        compiler_params=pltpu.CompilerParams(use_tc_tiling_on_sc=False),
        scratch_types=[
            pltpu.VMEM_SHARED((N, 8), jnp.float32),
            pltpu.VMEM((2, W_A, 8), jnp.float32),
            pltpu.VMEM((2, CKW_A * W_A), jnp.int32),
            pltpu.SemaphoreType.DMA,
            pltpu.SemaphoreType.DMA,
            pltpu.SemaphoreType.DMA,
        ],
    )


def _spmm_body(hl_hbm, hr_hbm, src_hbm, dst_hbm, z16_hbm, out_hbm,
               acc, rows_v, sidx_v, didx_v, gsem, ssem, isem):
    c = lax.axis_index("c")
    s = lax.axis_index("s")
    pltpu.sync_copy(z16_hbm, acc.at[pl.ds(s * ZCH, ZCH)])
    plsc.subcore_barrier()
    base_edge = s * (NWIN_B * W_B)

    def load_idx(w):
        e0 = base_edge + w * W_B
        pltpu.async_copy(src_hbm.at[pl.ds(e0, W_B)],
                         sidx_v.at[lax.rem(w, 2)], isem)
        pltpu.async_copy(dst_hbm.at[pl.ds(e0, W_B)],
                         didx_v.at[lax.rem(w, 3)], isem)

    def gather(w, b):
        sl = sidx_v.at[lax.rem(w, 2)]

        @pl.when(c == 0)
        def _():
            pltpu.async_copy(hl_hbm.at[sl], rows_v.at[b], gsem)

        @pl.when(c == 1)
        def _():
            pltpu.async_copy(hr_hbm.at[sl], rows_v.at[b], gsem)

    pltpu.sync_copy(src_hbm.at[pl.ds(base_edge, W_B)], sidx_v.at[0])
    pltpu.sync_copy(dst_hbm.at[pl.ds(base_edge, W_B)], didx_v.at[0])
    gather(0, 0)
    load_idx(1)

    def body(w, _):
        b = lax.rem(w, 2)
        nb = 1 - b

        @pl.when(w >= 1)
        def _():
            pltpu.make_async_copy(
                rows_v.at[nb],
                acc.at[didx_v.at[lax.rem(w - 1, 3)]],
                ssem).wait()

        @pl.when(w + 1 < NWIN_B)
        def _():
            pltpu.make_async_copy(src_hbm.at[pl.ds(0, W_B)], sidx_v.at[nb],
                                  isem).wait()
            pltpu.make_async_copy(dst_hbm.at[pl.ds(0, W_B)], didx_v.at[0],
                                  isem).wait()
            gather(w + 1, nb)

        pltpu.make_async_copy(hl_hbm.at[pl.ds(0, W_B)], rows_v.at[b],
                              gsem).wait()

        @pl.when(w + 2 < NWIN_B)
        def _():
            load_idx(w + 2)

        pltpu.async_copy(rows_v.at[b],
                         acc.at[didx_v.at[lax.rem(w, 3)]],
                         ssem, add=True)
        return 0

    lax.fori_loop(0, NWIN_B, body, 0)
    lb = (NWIN_B - 1) % 2
    pltpu.make_async_copy(
        rows_v.at[lb],
        acc.at[didx_v.at[lax.rem(NWIN_B - 1, 3)]],
        ssem).wait()
    plsc.subcore_barrier()
    pltpu.sync_copy(acc.at[pl.ds(s * ZCH, ZCH)], out_hbm.at[c, pl.ds(s * ZCH, ZCH)])


def _make_spmm_kernel():
    return pl.kernel(
        _spmm_body,
        out_type=jax.ShapeDtypeStruct((2, N, 16), jnp.float32),
        mesh=_sc_mesh(),
        compiler_params=pltpu.CompilerParams(use_tc_tiling_on_sc=False),
        scratch_types=[
            pltpu.VMEM_SHARED((N, 16), jnp.float32),
            pltpu.VMEM((2, W_B, 16), jnp.float32),
            pltpu.VMEM((2, W_B), jnp.int32),
            pltpu.VMEM((3, W_B), jnp.int32),
            pltpu.SemaphoreType.DMA,
            pltpu.SemaphoreType.DMA,
            pltpu.SemaphoreType.DMA,
        ],
    )


BLK = 800
NBLK = N // BLK


def _readin_body(state_ref, action_ref, wins_ref, wina_ref, b_ref, hl_ref, hr_ref):
    h = state_ref[...] @ wins_ref[...] + action_ref[...] @ wina_ref[...] + b_ref[...]
    h = jnp.where(h > 0, h, 0.01 * h)
    hl_ref[...] = h[:, :16]
    hr_ref[...] = h[:, 16:]


def _readin(state, action, w_s, w_a, b):
    return pl.pallas_call(
        _readin_body,
        grid=(NBLK,),
        in_specs=[
            pl.BlockSpec((BLK, 96), lambda i: (i, 0)),
            pl.BlockSpec((BLK, 32), lambda i: (i, 0)),
            pl.BlockSpec((96, 32), lambda i: (0, 0)),
            pl.BlockSpec((32, 32), lambda i: (0, 0)),
            pl.BlockSpec((1, 32), lambda i: (0, 0)),
        ],
        out_specs=[
            pl.BlockSpec((BLK, 16), lambda i: (i, 0)),
            pl.BlockSpec((BLK, 16), lambda i: (i, 0)),
        ],
        out_shape=[
            jax.ShapeDtypeStruct((N, 16), jnp.float32),
            jax.ShapeDtypeStruct((N, 16), jnp.float32),
        ],
    )(state, action, w_s, w_a, b)


def _layer_body(hl_ref, hr_ref, tl_ref, tr_ref, se0_ref, se1_ref,
                dg0_ref, dg1_ref, wself_ref, b_ref, a_ref, b4_ref,
                ol_ref, or_ref):
    h = jnp.concatenate([hl_ref[...], hr_ref[...]], axis=1)
    t = jnp.concatenate([tl_ref[...], tr_ref[...]], axis=1)
    se = se0_ref[...] + se1_ref[...]
    deg = jnp.clip(dg0_ref[...][:, :1] + dg1_ref[...][:, :1], 1.0, None)
    agg = (t @ a_ref[...] + se @ b4_ref[...]) / deg
    hn = h @ wself_ref[...] + b_ref[...] + agg
    hn = jnp.where(hn > 0, hn, 0.01 * hn)
    ol_ref[...] = hn[:, :16]
    or_ref[...] = hn[:, 16:]


def _layer(hl, hr, tl, tr, se0, se1, dg0, dg1, wself, b, a, b4):
    return pl.pallas_call(
        _layer_body,
        grid=(NBLK,),
        in_specs=[
            pl.BlockSpec((BLK, 16), lambda i: (i, 0)),
            pl.BlockSpec((BLK, 16), lambda i: (i, 0)),
            pl.BlockSpec((BLK, 16), lambda i: (i, 0)),
            pl.BlockSpec((BLK, 16), lambda i: (i, 0)),
            pl.BlockSpec((BLK, 4), lambda i: (i, 0)),
            pl.BlockSpec((BLK, 4), lambda i: (i, 0)),
            pl.BlockSpec((BLK, 4), lambda i: (i, 0)),
            pl.BlockSpec((BLK, 4), lambda i: (i, 0)),
            pl.BlockSpec((32, 32), lambda i: (0, 0)),
            pl.BlockSpec((1, 32), lambda i: (0, 0)),
            pl.BlockSpec((32, 32), lambda i: (0, 0)),
            pl.BlockSpec((4, 32), lambda i: (0, 0)),
        ],
        out_specs=[
            pl.BlockSpec((BLK, 16), lambda i: (i, 0)),
            pl.BlockSpec((BLK, 16), lambda i: (i, 0)),
        ],
        out_shape=[
            jax.ShapeDtypeStruct((N, 16), jnp.float32),
            jax.ShapeDtypeStruct((N, 16), jnp.float32),
        ],
    )(hl, hr, tl, tr, se0, se1, dg0, dg1, wself, b, a, b4)


def _readout_body(hl_ref, hr_ref, batch_ref, wout_ref, bout_ref, out_ref,
                  sums_ref, cnt_ref):
    i = pl.program_id(0)

    @pl.when(i == 0)
    def _():
        sums_ref[...] = jnp.zeros_like(sums_ref)
        cnt_ref[...] = jnp.zeros_like(cnt_ref)

    h = jnp.concatenate([hl_ref[...], hr_ref[...]], axis=1)
    y = h @ wout_ref[...] + bout_ref[...]
    gids = jax.lax.broadcasted_iota(jnp.int32, (1, G), 1)
    onehot = (batch_ref[...] == gids).astype(jnp.float32)
    sums_ref[...] += jnp.sum(onehot * y, axis=0, keepdims=True)
    cnt_ref[...] += jnp.sum(onehot, axis=0, keepdims=True)

    @pl.when(i == NBLK - 1)
    def _():
        out_ref[...] = sums_ref[...] / jnp.clip(cnt_ref[...], 1.0, None)


def _readout(hl, hr, batch_col, wout, bout):
    return pl.pallas_call(
        _readout_body,
        grid=(NBLK,),
        in_specs=[
            pl.BlockSpec((BLK, 16), lambda i: (i, 0)),
            pl.BlockSpec((BLK, 16), lambda i: (i, 0)),
            pl.BlockSpec((BLK, 1), lambda i: (i, 0)),
            pl.BlockSpec((32, 1), lambda i: (0, 0)),
            pl.BlockSpec((1, 1), lambda i: (0, 0)),
        ],
        out_specs=pl.BlockSpec((1, G), lambda i: (0, 0)),
        out_shape=jax.ShapeDtypeStruct((1, G), jnp.float32),
        scratch_shapes=[
            pltpu.VMEM((1, G), jnp.float32),
            pltpu.VMEM((1, G), jnp.float32),
        ],
    )(hl, hr, batch_col, wout, bout)


@jax.jit
def kernel(state, action, edge_index, edge_attr, batch,
           W_in, b_in, W_msg0, W_self0, b0, W_msg1, W_self1, b1, W_out, b_out):
    src = edge_index[0]
    dst = edge_index[1]

    ea8 = jnp.concatenate(
        [edge_attr,
         jnp.ones((E, 1), jnp.float32),
         jnp.zeros((E, 3), jnp.float32)], axis=1)
    z8 = jnp.zeros((ZCH, 8), jnp.float32)
    z16 = jnp.zeros((ZCH, 16), jnp.float32)

    sed = _make_sedeg_kernel()(ea8, dst, z8)
    se0 = sed[0, :, :4]
    se1 = sed[1, :, :4]
    dg0 = sed[0, :, 4:]
    dg1 = sed[1, :, 4:]

    hl, hr = _readin(state, action, W_in[:96], W_in[96:], b_in.reshape(1, C))

    spmm = _make_spmm_kernel()
    t = spmm(hl, hr, src, dst, z16)
    hl, hr = _layer(hl, hr, t[0], t[1], se0, se1, dg0, dg1,
                    W_self0, b0.reshape(1, C), W_msg0[:C], W_msg0[C:])

    t = spmm(hl, hr, src, dst, z16)
    hl, hr = _layer(hl, hr, t[0], t[1], se0, se1, dg0, dg1,
                    W_self1, b1.reshape(1, C), W_msg1[:C], W_msg1[C:])

    batch_col = batch.reshape(N, 1)
    out = _readout(hl, hr, batch_col, W_out, b_out.reshape(1, 1))
    return out.reshape(G, 1)


# speedup vs baseline: 7.5270x; 1.0154x over previous
import functools
import jax
import jax.numpy as jnp
from jax import lax
from jax.experimental import pallas as pl
from jax.experimental.pallas import tpu as pltpu
from jax.experimental.pallas import tpu_sc as plsc

N = 100000
E = 1600000
C = 32
G = 64

ZCH = N // 16

W_A = 1000
NWIN_A = 50
CKW_A = 10
NCH_A = NWIN_A // CKW_A
W_B = 800
NWIN_B = 125


def _sc_mesh():
    return plsc.VectorSubcoreMesh(core_axis_name="c", subcore_axis_name="s")


def _sedeg_body(ea8_hbm, ei_hbm, z8_hbm, out_hbm,
                acc8, ea_v, idx_v, lsem, ssem, isem):
    dst_hbm = ei_hbm.at[1]
    c = lax.axis_index("c")
    s = lax.axis_index("s")
    pltpu.sync_copy(z8_hbm, acc8.at[pl.ds(s * ZCH, ZCH)])
    t = c * 16 + s
    base_edge = t * (NWIN_A * W_A)
    plsc.subcore_barrier()

    def load_chunk(k):
        pltpu.async_copy(
            dst_hbm.at[pl.ds(base_edge + k * (CKW_A * W_A), CKW_A * W_A)],
            idx_v.at[lax.rem(k, 2)], isem)

    pltpu.sync_copy(dst_hbm.at[pl.ds(base_edge, CKW_A * W_A)], idx_v.at[0])
    load_chunk(1)

    def load(w, b):
        pltpu.async_copy(ea8_hbm.at[pl.ds(base_edge + w * W_A, W_A)],
                         ea_v.at[b], lsem)

    load(0, 0)
    dummy_idx = idx_v.at[0].at[pl.ds(0, W_A)]

    def body(w, _):
        b = lax.rem(w, 2)
        nb = 1 - b
        j = lax.rem(w, CKW_A)
        k = lax.div(w, CKW_A)
        idx_w = idx_v.at[lax.rem(k, 2)].at[pl.ds(j * W_A, W_A)]

        @pl.when(w >= 1)
        def _():
            pltpu.make_async_copy(ea_v.at[nb], acc8.at[dummy_idx],
                                  ssem).wait()

        @pl.when((j == 0) & (k >= 1))
        def _():
            pltpu.make_async_copy(dst_hbm.at[pl.ds(0, CKW_A * W_A)],
                                  idx_v.at[lax.rem(k, 2)], isem).wait()

        @pl.when((j == 0) & (k >= 1) & (k + 1 < NCH_A))
        def _():
            load_chunk(k + 1)

        @pl.when(w + 1 < NWIN_A)
        def _():
            load(w + 1, nb)

        pltpu.make_async_copy(ea8_hbm.at[pl.ds(0, W_A)], ea_v.at[b], lsem).wait()
        pltpu.async_copy(ea_v.at[b], acc8.at[idx_w], ssem, add=True)
        return 0

    lax.fori_loop(0, NWIN_A, body, 0)
    lb = (NWIN_A - 1) % 2
    pltpu.make_async_copy(ea_v.at[lb], acc8.at[dummy_idx], ssem).wait()
    plsc.subcore_barrier()
    pltpu.sync_copy(acc8.at[pl.ds(s * ZCH, ZCH)],
                    out_hbm.at[c, pl.ds(s * ZCH, ZCH)])


def _make_sedeg_kernel():
    return pl.kernel(
        _sedeg_body,
        out_type=jax.ShapeDtypeStruct((2, N, 8), jnp.float32),
        mesh=_sc_mesh(),
        compiler_params=pltpu.CompilerParams(use_tc_tiling_on_sc=False),
        scratch_types=[
            pltpu.VMEM_SHARED((N, 8), jnp.float32),
            pltpu.VMEM((2, W_A, 8), jnp.float32),
            pltpu.VMEM((2, CKW_A * W_A), jnp.int32),
            pltpu.SemaphoreType.DMA,
            pltpu.SemaphoreType.DMA,
            pltpu.SemaphoreType.DMA,
        ],
    )


def _spmm_body(hl_hbm, hr_hbm, ei_hbm, z16_hbm, out_hbm,
               acc, rows_v, sidx_v, didx_v, gsem, ssem, isem):
    src_hbm = ei_hbm.at[0]
    dst_hbm = ei_hbm.at[1]
    c = lax.axis_index("c")
    s = lax.axis_index("s")
    pltpu.sync_copy(z16_hbm, acc.at[pl.ds(s * ZCH, ZCH)])
    plsc.subcore_barrier()
    base_edge = s * (NWIN_B * W_B)

    def load_idx(w):
        e0 = base_edge + w * W_B
        pltpu.async_copy(src_hbm.at[pl.ds(e0, W_B)],
                         sidx_v.at[lax.rem(w, 2)], isem)
        pltpu.async_copy(dst_hbm.at[pl.ds(e0, W_B)],
                         didx_v.at[lax.rem(w, 3)], isem)

    def gather(w, b):
        sl = sidx_v.at[lax.rem(w, 2)]

        @pl.when(c == 0)
        def _():
            pltpu.async_copy(hl_hbm.at[sl], rows_v.at[b], gsem)

        @pl.when(c == 1)
        def _():
            pltpu.async_copy(hr_hbm.at[sl], rows_v.at[b], gsem)

    pltpu.sync_copy(src_hbm.at[pl.ds(base_edge, W_B)], sidx_v.at[0])
    pltpu.sync_copy(dst_hbm.at[pl.ds(base_edge, W_B)], didx_v.at[0])
    gather(0, 0)
    load_idx(1)

    def body(w, _):
        b = lax.rem(w, 2)
        nb = 1 - b

        @pl.when(w >= 1)
        def _():
            pltpu.make_async_copy(
                rows_v.at[nb],
                acc.at[didx_v.at[lax.rem(w - 1, 3)]],
                ssem).wait()

        @pl.when(w + 1 < NWIN_B)
        def _():
            pltpu.make_async_copy(src_hbm.at[pl.ds(0, W_B)], sidx_v.at[nb],
                                  isem).wait()
            pltpu.make_async_copy(dst_hbm.at[pl.ds(0, W_B)], didx_v.at[0],
                                  isem).wait()
            gather(w + 1, nb)

        pltpu.make_async_copy(hl_hbm.at[pl.ds(0, W_B)], rows_v.at[b],
                              gsem).wait()

        @pl.when(w + 2 < NWIN_B)
        def _():
            load_idx(w + 2)

        pltpu.async_copy(rows_v.at[b],
                         acc.at[didx_v.at[lax.rem(w, 3)]],
                         ssem, add=True)
        return 0

    lax.fori_loop(0, NWIN_B, body, 0)
    lb = (NWIN_B - 1) % 2
    pltpu.make_async_copy(
        rows_v.at[lb],
        acc.at[didx_v.at[lax.rem(NWIN_B - 1, 3)]],
        ssem).wait()
    plsc.subcore_barrier()
    pltpu.sync_copy(acc.at[pl.ds(s * ZCH, ZCH)], out_hbm.at[c, pl.ds(s * ZCH, ZCH)])


def _make_spmm_kernel():
    return pl.kernel(
        _spmm_body,
        out_type=jax.ShapeDtypeStruct((2, N, 16), jnp.float32),
        mesh=_sc_mesh(),
        compiler_params=pltpu.CompilerParams(use_tc_tiling_on_sc=False),
        scratch_types=[
            pltpu.VMEM_SHARED((N, 16), jnp.float32),
            pltpu.VMEM((2, W_B, 16), jnp.float32),
            pltpu.VMEM((2, W_B), jnp.int32),
            pltpu.VMEM((3, W_B), jnp.int32),
            pltpu.SemaphoreType.DMA,
            pltpu.SemaphoreType.DMA,
            pltpu.SemaphoreType.DMA,
        ],
    )


BLK = 800
NBLK = N // BLK


def _readin_body(state_ref, action_ref, wins_ref, wina_ref, b_ref, hl_ref, hr_ref):
    h = state_ref[...] @ wins_ref[...] + action_ref[...] @ wina_ref[...] + b_ref[...]
    h = jnp.where(h > 0, h, 0.01 * h)
    hl_ref[...] = h[:, :16]
    hr_ref[...] = h[:, 16:]


def _readin(state, action, w_s, w_a, b):
    return pl.pallas_call(
        _readin_body,
        grid=(NBLK,),
        in_specs=[
            pl.BlockSpec((BLK, 96), lambda i: (i, 0)),
            pl.BlockSpec((BLK, 32), lambda i: (i, 0)),
            pl.BlockSpec((96, 32), lambda i: (0, 0)),
            pl.BlockSpec((32, 32), lambda i: (0, 0)),
            pl.BlockSpec((1, 32), lambda i: (0, 0)),
        ],
        out_specs=[
            pl.BlockSpec((BLK, 16), lambda i: (i, 0)),
            pl.BlockSpec((BLK, 16), lambda i: (i, 0)),
        ],
        out_shape=[
            jax.ShapeDtypeStruct((N, 16), jnp.float32),
            jax.ShapeDtypeStruct((N, 16), jnp.float32),
        ],
    )(state, action, w_s, w_a, b)


def _layer_body(hl_ref, hr_ref, tl_ref, tr_ref, se0_ref, se1_ref,
                dg0_ref, dg1_ref, wself_ref, b_ref, a_ref, b4_ref,
                ol_ref, or_ref):
    h = jnp.concatenate([hl_ref[...], hr_ref[...]], axis=1)
    t = jnp.concatenate([tl_ref[...], tr_ref[...]], axis=1)
    se = se0_ref[...] + se1_ref[...]
    deg = jnp.clip(dg0_ref[...][:, :1] + dg1_ref[...][:, :1], 1.0, None)
    agg = (t @ a_ref[...] + se @ b4_ref[...]) / deg
    hn = h @ wself_ref[...] + b_ref[...] + agg
    hn = jnp.where(hn > 0, hn, 0.01 * hn)
    ol_ref[...] = hn[:, :16]
    or_ref[...] = hn[:, 16:]


def _layer(hl, hr, tl, tr, se0, se1, dg0, dg1, wself, b, a, b4):
    return pl.pallas_call(
        _layer_body,
        grid=(NBLK,),
        in_specs=[
            pl.BlockSpec((BLK, 16), lambda i: (i, 0)),
            pl.BlockSpec((BLK, 16), lambda i: (i, 0)),
            pl.BlockSpec((BLK, 16), lambda i: (i, 0)),
            pl.BlockSpec((BLK, 16), lambda i: (i, 0)),
            pl.BlockSpec((BLK, 4), lambda i: (i, 0)),
            pl.BlockSpec((BLK, 4), lambda i: (i, 0)),
            pl.BlockSpec((BLK, 4), lambda i: (i, 0)),
            pl.BlockSpec((BLK, 4), lambda i: (i, 0)),
            pl.BlockSpec((32, 32), lambda i: (0, 0)),
            pl.BlockSpec((1, 32), lambda i: (0, 0)),
            pl.BlockSpec((32, 32), lambda i: (0, 0)),
            pl.BlockSpec((4, 32), lambda i: (0, 0)),
        ],
        out_specs=[
            pl.BlockSpec((BLK, 16), lambda i: (i, 0)),
            pl.BlockSpec((BLK, 16), lambda i: (i, 0)),
        ],
        out_shape=[
            jax.ShapeDtypeStruct((N, 16), jnp.float32),
            jax.ShapeDtypeStruct((N, 16), jnp.float32),
        ],
    )(hl, hr, tl, tr, se0, se1, dg0, dg1, wself, b, a, b4)


def _readout_body(hl_ref, hr_ref, batch_ref, wout_ref, bout_ref, out_ref,
                  sums_ref, cnt_ref):
    i = pl.program_id(0)

    @pl.when(i == 0)
    def _():
        sums_ref[...] = jnp.zeros_like(sums_ref)
        cnt_ref[...] = jnp.zeros_like(cnt_ref)

    h = jnp.concatenate([hl_ref[...], hr_ref[...]], axis=1)
    y = h @ wout_ref[...] + bout_ref[...]
    gids = jax.lax.broadcasted_iota(jnp.int32, (1, G), 1)
    onehot = (batch_ref[...] == gids).astype(jnp.float32)
    sums_ref[...] += jnp.sum(onehot * y, axis=0, keepdims=True)
    cnt_ref[...] += jnp.sum(onehot, axis=0, keepdims=True)

    @pl.when(i == NBLK - 1)
    def _():
        out_ref[...] = sums_ref[...] / jnp.clip(cnt_ref[...], 1.0, None)


def _readout(hl, hr, batch_col, wout, bout):
    return pl.pallas_call(
        _readout_body,
        grid=(NBLK,),
        in_specs=[
            pl.BlockSpec((BLK, 16), lambda i: (i, 0)),
            pl.BlockSpec((BLK, 16), lambda i: (i, 0)),
            pl.BlockSpec((BLK, 1), lambda i: (i, 0)),
            pl.BlockSpec((32, 1), lambda i: (0, 0)),
            pl.BlockSpec((1, 1), lambda i: (0, 0)),
        ],
        out_specs=pl.BlockSpec((1, G), lambda i: (0, 0)),
        out_shape=jax.ShapeDtypeStruct((1, G), jnp.float32),
        scratch_shapes=[
            pltpu.VMEM((1, G), jnp.float32),
            pltpu.VMEM((1, G), jnp.float32),
        ],
    )(hl, hr, batch_col, wout, bout)


@jax.jit
def kernel(state, action, edge_index, edge_attr, batch,
           W_in, b_in, W_msg0, W_self0, b0, W_msg1, W_self1, b1, W_out, b_out):
    ea8 = jnp.concatenate(
        [edge_attr,
         jnp.ones((E, 1), jnp.float32),
         jnp.zeros((E, 3), jnp.float32)], axis=1)
    z8 = jnp.zeros((ZCH, 8), jnp.float32)
    z16 = jnp.zeros((ZCH, 16), jnp.float32)

    sed = _make_sedeg_kernel()(ea8, edge_index, z8)
    se0 = sed[0, :, :4]
    se1 = sed[1, :, :4]
    dg0 = sed[0, :, 4:]
    dg1 = sed[1, :, 4:]

    hl, hr = _readin(state, action, W_in[:96], W_in[96:], b_in.reshape(1, C))

    spmm = _make_spmm_kernel()
    t = spmm(hl, hr, edge_index, z16)
    hl, hr = _layer(hl, hr, t[0], t[1], se0, se1, dg0, dg1,
                    W_self0, b0.reshape(1, C), W_msg0[:C], W_msg0[C:])

    t = spmm(hl, hr, edge_index, z16)
    hl, hr = _layer(hl, hr, t[0], t[1], se0, se1, dg0, dg1,
                    W_self1, b1.reshape(1, C), W_msg1[:C], W_msg1[C:])

    batch_col = batch.reshape(N, 1)
    out = _readout(hl, hr, batch_col, W_out, b_out.reshape(1, 1))
    return out.reshape(G, 1)


# speedup vs baseline: 8.7392x; 1.1611x over previous
import functools
import jax
import jax.numpy as jnp
from jax import lax
from jax.experimental import pallas as pl
from jax.experimental.pallas import tpu as pltpu
from jax.experimental.pallas import tpu_sc as plsc

N = 100000
E = 1600000
C = 32
G = 64

ZCH = N // 16

W_A = 1000
NWIN_A = 50
CKW_A = 10
NCH_A = NWIN_A // CKW_A
W_B = 800
NWIN_B = 125


def _sc_mesh():
    return plsc.VectorSubcoreMesh(core_axis_name="c", subcore_axis_name="s")


def _sedeg_body(ea8_hbm, ei_hbm, z8_hbm, out_hbm,
                acc8, ea_v, idx_v, lsem, ssem, isem):
    dst_hbm = ei_hbm.at[1]
    c = lax.axis_index("c")
    s = lax.axis_index("s")
    pltpu.sync_copy(z8_hbm, acc8.at[pl.ds(s * ZCH, ZCH)])
    t = c * 16 + s
    base_edge = t * (NWIN_A * W_A)
    plsc.subcore_barrier()

    def load_chunk(k):
        pltpu.async_copy(
            dst_hbm.at[pl.ds(base_edge + k * (CKW_A * W_A), CKW_A * W_A)],
            idx_v.at[lax.rem(k, 2)], isem)

    pltpu.sync_copy(dst_hbm.at[pl.ds(base_edge, CKW_A * W_A)], idx_v.at[0])
    load_chunk(1)

    def load(w, b):
        pltpu.async_copy(ea8_hbm.at[pl.ds(base_edge + w * W_A, W_A)],
                         ea_v.at[b], lsem)

    load(0, 0)
    dummy_idx = idx_v.at[0].at[pl.ds(0, W_A)]

    def body(w, _):
        b = lax.rem(w, 2)
        nb = 1 - b
        j = lax.rem(w, CKW_A)
        k = lax.div(w, CKW_A)
        idx_w = idx_v.at[lax.rem(k, 2)].at[pl.ds(j * W_A, W_A)]

        @pl.when(w >= 1)
        def _():
            pltpu.make_async_copy(ea_v.at[nb], acc8.at[dummy_idx],
                                  ssem).wait()

        @pl.when((j == 0) & (k >= 1))
        def _():
            pltpu.make_async_copy(dst_hbm.at[pl.ds(0, CKW_A * W_A)],
                                  idx_v.at[lax.rem(k, 2)], isem).wait()

        @pl.when((j == 0) & (k >= 1) & (k + 1 < NCH_A))
        def _():
            load_chunk(k + 1)

        @pl.when(w + 1 < NWIN_A)
        def _():
            load(w + 1, nb)

        pltpu.make_async_copy(ea8_hbm.at[pl.ds(0, W_A)], ea_v.at[b], lsem).wait()
        pltpu.async_copy(ea_v.at[b], acc8.at[idx_w], ssem, add=True)
        return 0

    lax.fori_loop(0, NWIN_A, body, 0)
    lb = (NWIN_A - 1) % 2
    pltpu.make_async_copy(ea_v.at[lb], acc8.at[dummy_idx], ssem).wait()
    plsc.subcore_barrier()
    pltpu.sync_copy(acc8.at[pl.ds(s * ZCH, ZCH)],
                    out_hbm.at[c, pl.ds(s * ZCH, ZCH)])


def _make_sedeg_kernel():
    return pl.kernel(
        _sedeg_body,
        out_type=jax.ShapeDtypeStruct((2, N, 8), jnp.float32),
        mesh=_sc_mesh(),
        compiler_params=pltpu.CompilerParams(use_tc_tiling_on_sc=False),
        scratch_types=[
            pltpu.VMEM_SHARED((N, 8), jnp.float32),
            pltpu.VMEM((2, W_A, 8), jnp.float32),
            pltpu.VMEM((2, CKW_A * W_A), jnp.int32),
            pltpu.SemaphoreType.DMA,
            pltpu.SemaphoreType.DMA,
            pltpu.SemaphoreType.DMA,
        ],
    )


def _spmm_body(hl_hbm, hr_hbm, ei_hbm, z16_hbm, out_hbm,
               acc, rows_v, sidx_v, didx_v, gsem, ssem, isem):
    src_hbm = ei_hbm.at[0]
    dst_hbm = ei_hbm.at[1]
    c = lax.axis_index("c")
    s = lax.axis_index("s")
    pltpu.sync_copy(z16_hbm, acc.at[pl.ds(s * ZCH, ZCH)])
    plsc.subcore_barrier()
    base_edge = s * (NWIN_B * W_B)

    def load_idx(w):
        e0 = base_edge + w * W_B
        pltpu.async_copy(src_hbm.at[pl.ds(e0, W_B)],
                         sidx_v.at[lax.rem(w, 2)], isem)
        pltpu.async_copy(dst_hbm.at[pl.ds(e0, W_B)],
                         didx_v.at[lax.rem(w, 3)], isem)

    def gather(w, b):
        sl = sidx_v.at[lax.rem(w, 2)]

        @pl.when(c == 0)
        def _():
            pltpu.async_copy(hl_hbm.at[sl], rows_v.at[b], gsem)

        @pl.when(c == 1)
        def _():
            pltpu.async_copy(hr_hbm.at[sl], rows_v.at[b], gsem)

    pltpu.sync_copy(src_hbm.at[pl.ds(base_edge, W_B)], sidx_v.at[0])
    pltpu.sync_copy(dst_hbm.at[pl.ds(base_edge, W_B)], didx_v.at[0])
    gather(0, 0)
    load_idx(1)

    def body(w, _):
        b = lax.rem(w, 2)
        nb = 1 - b

        @pl.when(w >= 1)
        def _():
            pltpu.make_async_copy(
                rows_v.at[nb],
                acc.at[didx_v.at[lax.rem(w - 1, 3)]],
                ssem).wait()

        @pl.when(w + 1 < NWIN_B)
        def _():
            pltpu.make_async_copy(src_hbm.at[pl.ds(0, W_B)], sidx_v.at[nb],
                                  isem).wait()
            pltpu.make_async_copy(dst_hbm.at[pl.ds(0, W_B)], didx_v.at[0],
                                  isem).wait()
            gather(w + 1, nb)

        pltpu.make_async_copy(hl_hbm.at[pl.ds(0, W_B)], rows_v.at[b],
                              gsem).wait()

        @pl.when(w + 2 < NWIN_B)
        def _():
            load_idx(w + 2)

        pltpu.async_copy(rows_v.at[b],
                         acc.at[didx_v.at[lax.rem(w, 3)]],
                         ssem, add=True)
        return 0

    lax.fori_loop(0, NWIN_B, body, 0)
    lb = (NWIN_B - 1) % 2
    pltpu.make_async_copy(
        rows_v.at[lb],
        acc.at[didx_v.at[lax.rem(NWIN_B - 1, 3)]],
        ssem).wait()
    plsc.subcore_barrier()
    pltpu.sync_copy(acc.at[pl.ds(s * ZCH, ZCH)], out_hbm.at[c, pl.ds(s * ZCH, ZCH)])


def _make_spmm_kernel():
    return pl.kernel(
        _spmm_body,
        out_type=jax.ShapeDtypeStruct((2, N, 16), jnp.float32),
        mesh=_sc_mesh(),
        compiler_params=pltpu.CompilerParams(use_tc_tiling_on_sc=False),
        scratch_types=[
            pltpu.VMEM_SHARED((N, 16), jnp.float32),
            pltpu.VMEM((2, W_B, 16), jnp.float32),
            pltpu.VMEM((2, W_B), jnp.int32),
            pltpu.VMEM((3, W_B), jnp.int32),
            pltpu.SemaphoreType.DMA,
            pltpu.SemaphoreType.DMA,
            pltpu.SemaphoreType.DMA,
        ],
    )


BLK = 800
NBLK = N // BLK


def _readin_body(state_ref, action_ref, wins_ref, wina_ref, b_ref, hl_ref, hr_ref):
    h = state_ref[...] @ wins_ref[...] + action_ref[...] @ wina_ref[...] + b_ref[...]
    h = jnp.where(h > 0, h, 0.01 * h)
    hl_ref[...] = h[:, :16]
    hr_ref[...] = h[:, 16:]


def _readin(state, action, w_s, w_a, b):
    return pl.pallas_call(
        _readin_body,
        grid=(NBLK,),
        in_specs=[
            pl.BlockSpec((BLK, 96), lambda i: (i, 0)),
            pl.BlockSpec((BLK, 32), lambda i: (i, 0)),
            pl.BlockSpec((96, 32), lambda i: (0, 0)),
            pl.BlockSpec((32, 32), lambda i: (0, 0)),
            pl.BlockSpec((1, 32), lambda i: (0, 0)),
        ],
        out_specs=[
            pl.BlockSpec((BLK, 16), lambda i: (i, 0)),
            pl.BlockSpec((BLK, 16), lambda i: (i, 0)),
        ],
        out_shape=[
            jax.ShapeDtypeStruct((N, 16), jnp.float32),
            jax.ShapeDtypeStruct((N, 16), jnp.float32),
        ],
    )(state, action, w_s, w_a, b)


def _layer_body(hl_ref, hr_ref, t0_ref, t1_ref, s0_ref, s1_ref,
                wself_ref, b_ref, a_ref, b4_ref, ol_ref, or_ref):
    h = jnp.concatenate([hl_ref[...], hr_ref[...]], axis=1)
    t = jnp.concatenate([t0_ref[0], t1_ref[0]], axis=1)
    se = s0_ref[0, :, :4] + s1_ref[0, :, :4]
    deg = jnp.clip(s0_ref[0, :, 4:5] + s1_ref[0, :, 4:5], 1.0, None)
    agg = (t @ a_ref[...] + se @ b4_ref[...]) / deg
    hn = h @ wself_ref[...] + b_ref[...] + agg
    hn = jnp.where(hn > 0, hn, 0.01 * hn)
    ol_ref[...] = hn[:, :16]
    or_ref[...] = hn[:, 16:]


def _layer(hl, hr, t, sed, wself, b, a, b4):
    return pl.pallas_call(
        _layer_body,
        grid=(NBLK,),
        in_specs=[
            pl.BlockSpec((BLK, 16), lambda i: (i, 0)),
            pl.BlockSpec((BLK, 16), lambda i: (i, 0)),
            pl.BlockSpec((1, BLK, 16), lambda i: (0, i, 0)),
            pl.BlockSpec((1, BLK, 16), lambda i: (1, i, 0)),
            pl.BlockSpec((1, BLK, 8), lambda i: (0, i, 0)),
            pl.BlockSpec((1, BLK, 8), lambda i: (1, i, 0)),
            pl.BlockSpec((32, 32), lambda i: (0, 0)),
            pl.BlockSpec((1, 32), lambda i: (0, 0)),
            pl.BlockSpec((32, 32), lambda i: (0, 0)),
            pl.BlockSpec((4, 32), lambda i: (0, 0)),
        ],
        out_specs=[
            pl.BlockSpec((BLK, 16), lambda i: (i, 0)),
            pl.BlockSpec((BLK, 16), lambda i: (i, 0)),
        ],
        out_shape=[
            jax.ShapeDtypeStruct((N, 16), jnp.float32),
            jax.ShapeDtypeStruct((N, 16), jnp.float32),
        ],
    )(hl, hr, t, t, sed, sed, wself, b, a, b4)


def _readout_body(hl_ref, hr_ref, batch_ref, wout_ref, bout_ref, out_ref,
                  sums_ref, cnt_ref):
    i = pl.program_id(0)

    @pl.when(i == 0)
    def _():
        sums_ref[...] = jnp.zeros_like(sums_ref)
        cnt_ref[...] = jnp.zeros_like(cnt_ref)

    h = jnp.concatenate([hl_ref[...], hr_ref[...]], axis=1)
    y = h @ wout_ref[...] + bout_ref[...]
    gids = jax.lax.broadcasted_iota(jnp.int32, (1, G), 1)
    onehot = (batch_ref[...] == gids).astype(jnp.float32)
    sums_ref[...] += jnp.sum(onehot * y, axis=0, keepdims=True)
    cnt_ref[...] += jnp.sum(onehot, axis=0, keepdims=True)

    @pl.when(i == NBLK - 1)
    def _():
        out_ref[...] = sums_ref[...] / jnp.clip(cnt_ref[...], 1.0, None)


def _readout(hl, hr, batch_col, wout, bout):
    return pl.pallas_call(
        _readout_body,
        grid=(NBLK,),
        in_specs=[
            pl.BlockSpec((BLK, 16), lambda i: (i, 0)),
            pl.BlockSpec((BLK, 16), lambda i: (i, 0)),
            pl.BlockSpec((BLK, 1), lambda i: (i, 0)),
            pl.BlockSpec((32, 1), lambda i: (0, 0)),
            pl.BlockSpec((1, 1), lambda i: (0, 0)),
        ],
        out_specs=pl.BlockSpec((1, G), lambda i: (0, 0)),
        out_shape=jax.ShapeDtypeStruct((1, G), jnp.float32),
        scratch_shapes=[
            pltpu.VMEM((1, G), jnp.float32),
            pltpu.VMEM((1, G), jnp.float32),
        ],
    )(hl, hr, batch_col, wout, bout)


@jax.jit
def kernel(state, action, edge_index, edge_attr, batch,
           W_in, b_in, W_msg0, W_self0, b0, W_msg1, W_self1, b1, W_out, b_out):
    ea8 = jnp.concatenate(
        [edge_attr,
         jnp.ones((E, 1), jnp.float32),
         jnp.zeros((E, 3), jnp.float32)], axis=1)
    z8 = jnp.zeros((ZCH, 8), jnp.float32)
    z16 = jnp.zeros((ZCH, 16), jnp.float32)

    sed = _make_sedeg_kernel()(ea8, edge_index, z8)

    hl, hr = _readin(state, action, W_in[:96], W_in[96:], b_in.reshape(1, C))

    spmm = _make_spmm_kernel()
    t = spmm(hl, hr, edge_index, z16)
    hl, hr = _layer(hl, hr, t, sed,
                    W_self0, b0.reshape(1, C), W_msg0[:C], W_msg0[C:])

    t = spmm(hl, hr, edge_index, z16)
    hl, hr = _layer(hl, hr, t, sed,
                    W_self1, b1.reshape(1, C), W_msg1[:C], W_msg1[C:])

    batch_col = batch.reshape(N, 1)
    out = _readout(hl, hr, batch_col, W_out, b_out.reshape(1, 1))
    return out.reshape(G, 1)


# speedup vs baseline: 9.3303x; 1.0676x over previous
import functools
import jax
import jax.numpy as jnp
from jax import lax
from jax.experimental import pallas as pl
from jax.experimental.pallas import tpu as pltpu
from jax.experimental.pallas import tpu_sc as plsc

N = 100000
E = 1600000
C = 32
G = 64

ZCH = N // 16

W_A = 1000
NWIN_A = 50
CKW_A = 10
NCH_A = NWIN_A // CKW_A
W_B = 800
NWIN_B = 125


def _sc_mesh():
    return plsc.VectorSubcoreMesh(core_axis_name="c", subcore_axis_name="s")


def _sedeg_body(ea8_hbm, ei_hbm, z8_hbm, out_hbm,
                acc8, ea_v, idx_v, lsem, ssem, isem):
    dst_hbm = ei_hbm.at[1]
    c = lax.axis_index("c")
    s = lax.axis_index("s")
    pltpu.sync_copy(z8_hbm, acc8.at[pl.ds(s * ZCH, ZCH)])
    t = c * 16 + s
    base_edge = t * (NWIN_A * W_A)
    plsc.subcore_barrier()

    def load_chunk(k):
        pltpu.async_copy(
            dst_hbm.at[pl.ds(base_edge + k * (CKW_A * W_A), CKW_A * W_A)],
            idx_v.at[lax.rem(k, 2)], isem)

    pltpu.sync_copy(dst_hbm.at[pl.ds(base_edge, CKW_A * W_A)], idx_v.at[0])
    load_chunk(1)

    def load(w, b):
        pltpu.async_copy(ea8_hbm.at[pl.ds(base_edge + w * W_A, W_A)],
                         ea_v.at[b], lsem)

    load(0, 0)
    dummy_idx = idx_v.at[0].at[pl.ds(0, W_A)]

    def body(w, _):
        b = lax.rem(w, 2)
        nb = 1 - b
        j = lax.rem(w, CKW_A)
        k = lax.div(w, CKW_A)
        idx_w = idx_v.at[lax.rem(k, 2)].at[pl.ds(j * W_A, W_A)]

        @pl.when(w >= 1)
        def _():
            pltpu.make_async_copy(ea_v.at[nb], acc8.at[dummy_idx],
                                  ssem).wait()

        @pl.when((j == 0) & (k >= 1))
        def _():
            pltpu.make_async_copy(dst_hbm.at[pl.ds(0, CKW_A * W_A)],
                                  idx_v.at[lax.rem(k, 2)], isem).wait()

        @pl.when((j == 0) & (k >= 1) & (k + 1 < NCH_A))
        def _():
            load_chunk(k + 1)

        @pl.when(w + 1 < NWIN_A)
        def _():
            load(w + 1, nb)

        pltpu.make_async_copy(ea8_hbm.at[pl.ds(0, W_A)], ea_v.at[b], lsem).wait()
        pltpu.async_copy(ea_v.at[b], acc8.at[idx_w], ssem, add=True)
        return 0

    lax.fori_loop(0, NWIN_A, body, 0)
    lb = (NWIN_A - 1) % 2
    pltpu.make_async_copy(ea_v.at[lb], acc8.at[dummy_idx], ssem).wait()
    plsc.subcore_barrier()
    pltpu.sync_copy(acc8.at[pl.ds(s * ZCH, ZCH)],
                    out_hbm.at[c, pl.ds(s * ZCH, ZCH)])


def _make_sedeg_kernel():
    return pl.kernel(
        _sedeg_body,
        out_type=jax.ShapeDtypeStruct((2, N, 8), jnp.float32),
        mesh=_sc_mesh(),
        compiler_params=pltpu.CompilerParams(use_tc_tiling_on_sc=False),
        scratch_types=[
            pltpu.VMEM_SHARED((N, 8), jnp.float32),
            pltpu.VMEM((2, W_A, 8), jnp.float32),
            pltpu.VMEM((2, CKW_A * W_A), jnp.int32),
            pltpu.SemaphoreType.DMA,
            pltpu.SemaphoreType.DMA,
            pltpu.SemaphoreType.DMA,
        ],
    )


def _spmm_body(hl_hbm, hr_hbm, ei_hbm, z16_hbm, out_hbm,
               acc, rows_v, sidx_v, didx_v, gsem, ssem, isem):
    src_hbm = ei_hbm.at[0]
    dst_hbm = ei_hbm.at[1]
    c = lax.axis_index("c")
    s = lax.axis_index("s")
    pltpu.sync_copy(z16_hbm, acc.at[pl.ds(s * ZCH, ZCH)])
    plsc.subcore_barrier()
    base_edge = s * (NWIN_B * W_B)

    def load_idx(w):
        e0 = base_edge + w * W_B
        pltpu.async_copy(src_hbm.at[pl.ds(e0, W_B)],
                         sidx_v.at[lax.rem(w, 2)], isem)
        pltpu.async_copy(dst_hbm.at[pl.ds(e0, W_B)],
                         didx_v.at[lax.rem(w, 3)], isem)

    def gather(w, b):
        sl = sidx_v.at[lax.rem(w, 2)]

        @pl.when(c == 0)
        def _():
            pltpu.async_copy(hl_hbm.at[sl], rows_v.at[b], gsem)

        @pl.when(c == 1)
        def _():
            pltpu.async_copy(hr_hbm.at[sl], rows_v.at[b], gsem)

    pltpu.sync_copy(src_hbm.at[pl.ds(base_edge, W_B)], sidx_v.at[0])
    pltpu.sync_copy(dst_hbm.at[pl.ds(base_edge, W_B)], didx_v.at[0])
    gather(0, 0)
    load_idx(1)

    def body(w, _):
        b = lax.rem(w, 2)
        nb = 1 - b

        @pl.when(w >= 1)
        def _():
            pltpu.make_async_copy(
                rows_v.at[nb],
                acc.at[didx_v.at[lax.rem(w - 1, 3)]],
                ssem).wait()

        @pl.when(w + 1 < NWIN_B)
        def _():
            pltpu.make_async_copy(src_hbm.at[pl.ds(0, W_B)], sidx_v.at[nb],
                                  isem).wait()
            pltpu.make_async_copy(dst_hbm.at[pl.ds(0, W_B)], didx_v.at[0],
                                  isem).wait()
            gather(w + 1, nb)

        pltpu.make_async_copy(hl_hbm.at[pl.ds(0, W_B)], rows_v.at[b],
                              gsem).wait()

        @pl.when(w + 2 < NWIN_B)
        def _():
            load_idx(w + 2)

        pltpu.async_copy(rows_v.at[b],
                         acc.at[didx_v.at[lax.rem(w, 3)]],
                         ssem, add=True)
        return 0

    lax.fori_loop(0, NWIN_B, body, 0)
    lb = (NWIN_B - 1) % 2
    pltpu.make_async_copy(
        rows_v.at[lb],
        acc.at[didx_v.at[lax.rem(NWIN_B - 1, 3)]],
        ssem).wait()
    plsc.subcore_barrier()
    pltpu.sync_copy(acc.at[pl.ds(s * ZCH, ZCH)], out_hbm.at[c, pl.ds(s * ZCH, ZCH)])


def _make_spmm_kernel():
    return pl.kernel(
        _spmm_body,
        out_type=jax.ShapeDtypeStruct((2, N, 16), jnp.float32),
        mesh=_sc_mesh(),
        compiler_params=pltpu.CompilerParams(use_tc_tiling_on_sc=False),
        scratch_types=[
            pltpu.VMEM_SHARED((N, 16), jnp.float32),
            pltpu.VMEM((2, W_B, 16), jnp.float32),
            pltpu.VMEM((2, W_B), jnp.int32),
            pltpu.VMEM((3, W_B), jnp.int32),
            pltpu.SemaphoreType.DMA,
            pltpu.SemaphoreType.DMA,
            pltpu.SemaphoreType.DMA,
        ],
    )


BLK = 800
NBLK = N // BLK


def _readin_body(state_ref, action_ref, wins_ref, wina_ref, b_ref, hl_ref, hr_ref):
    h = state_ref[...] @ wins_ref[...] + action_ref[...] @ wina_ref[...] + b_ref[...]
    h = jnp.where(h > 0, h, 0.01 * h)
    hl_ref[...] = h[:, :16]
    hr_ref[...] = h[:, 16:]


def _readin(state, action, w_s, w_a, b):
    return pl.pallas_call(
        _readin_body,
        grid=(NBLK,),
        in_specs=[
            pl.BlockSpec((BLK, 96), lambda i: (i, 0)),
            pl.BlockSpec((BLK, 32), lambda i: (i, 0)),
            pl.BlockSpec((96, 32), lambda i: (0, 0)),
            pl.BlockSpec((32, 32), lambda i: (0, 0)),
            pl.BlockSpec((1, 32), lambda i: (0, 0)),
        ],
        out_specs=[
            pl.BlockSpec((BLK, 16), lambda i: (i, 0)),
            pl.BlockSpec((BLK, 16), lambda i: (i, 0)),
        ],
        out_shape=[
            jax.ShapeDtypeStruct((N, 16), jnp.float32),
            jax.ShapeDtypeStruct((N, 16), jnp.float32),
        ],
    )(state, action, w_s, w_a, b)


def _layer_body(hl_ref, hr_ref, t0_ref, t1_ref, s0_ref, s1_ref,
                wself_ref, b_ref, a_ref, b4_ref, ol_ref, or_ref):
    h = jnp.concatenate([hl_ref[...], hr_ref[...]], axis=1)
    t = jnp.concatenate([t0_ref[0], t1_ref[0]], axis=1)
    se = s0_ref[0, :, :4] + s1_ref[0, :, :4]
    deg = jnp.clip(s0_ref[0, :, 4:5] + s1_ref[0, :, 4:5], 1.0, None)
    agg = (t @ a_ref[...] + se @ b4_ref[...]) / deg
    hn = h @ wself_ref[...] + b_ref[...] + agg
    hn = jnp.where(hn > 0, hn, 0.01 * hn)
    ol_ref[...] = hn[:, :16]
    or_ref[...] = hn[:, 16:]


def _layer(hl, hr, t, sed, wself, b, a, b4):
    return pl.pallas_call(
        _layer_body,
        grid=(NBLK,),
        in_specs=[
            pl.BlockSpec((BLK, 16), lambda i: (i, 0)),
            pl.BlockSpec((BLK, 16), lambda i: (i, 0)),
            pl.BlockSpec((1, BLK, 16), lambda i: (0, i, 0)),
            pl.BlockSpec((1, BLK, 16), lambda i: (1, i, 0)),
            pl.BlockSpec((1, BLK, 8), lambda i: (0, i, 0)),
            pl.BlockSpec((1, BLK, 8), lambda i: (1, i, 0)),
            pl.BlockSpec((32, 32), lambda i: (0, 0)),
            pl.BlockSpec((1, 32), lambda i: (0, 0)),
            pl.BlockSpec((32, 32), lambda i: (0, 0)),
            pl.BlockSpec((4, 32), lambda i: (0, 0)),
        ],
        out_specs=[
            pl.BlockSpec((BLK, 16), lambda i: (i, 0)),
            pl.BlockSpec((BLK, 16), lambda i: (i, 0)),
        ],
        out_shape=[
            jax.ShapeDtypeStruct((N, 16), jnp.float32),
            jax.ShapeDtypeStruct((N, 16), jnp.float32),
        ],
    )(hl, hr, t, t, sed, sed, wself, b, a, b4)


def _layer_ro_body(hl_ref, hr_ref, t0_ref, t1_ref, s0_ref, s1_ref,
                   wself_ref, b_ref, a_ref, b4_ref, batch_ref,
                   wout_ref, bout_ref, out_ref, sums_ref, cnt_ref):
    i = pl.program_id(0)

    @pl.when(i == 0)
    def _():
        sums_ref[...] = jnp.zeros_like(sums_ref)
        cnt_ref[...] = jnp.zeros_like(cnt_ref)

    h = jnp.concatenate([hl_ref[...], hr_ref[...]], axis=1)
    t = jnp.concatenate([t0_ref[0], t1_ref[0]], axis=1)
    se = s0_ref[0, :, :4] + s1_ref[0, :, :4]
    deg = jnp.clip(s0_ref[0, :, 4:5] + s1_ref[0, :, 4:5], 1.0, None)
    agg = (t @ a_ref[...] + se @ b4_ref[...]) / deg
    hn = h @ wself_ref[...] + b_ref[...] + agg
    hn = jnp.where(hn > 0, hn, 0.01 * hn)
    y = hn @ wout_ref[...] + bout_ref[...]
    gids = jax.lax.broadcasted_iota(jnp.int32, (1, G), 1)
    onehot = (batch_ref[...] == gids).astype(jnp.float32)
    sums_ref[...] += jnp.sum(onehot * y, axis=0, keepdims=True)
    cnt_ref[...] += jnp.sum(onehot, axis=0, keepdims=True)

    @pl.when(i == NBLK - 1)
    def _():
        out_ref[...] = sums_ref[...] / jnp.clip(cnt_ref[...], 1.0, None)


def _layer_readout(hl, hr, t, sed, wself, b, a, b4, batch_col, wout, bout):
    return pl.pallas_call(
        _layer_ro_body,
        grid=(NBLK,),
        in_specs=[
            pl.BlockSpec((BLK, 16), lambda i: (i, 0)),
            pl.BlockSpec((BLK, 16), lambda i: (i, 0)),
            pl.BlockSpec((1, BLK, 16), lambda i: (0, i, 0)),
            pl.BlockSpec((1, BLK, 16), lambda i: (1, i, 0)),
            pl.BlockSpec((1, BLK, 8), lambda i: (0, i, 0)),
            pl.BlockSpec((1, BLK, 8), lambda i: (1, i, 0)),
            pl.BlockSpec((32, 32), lambda i: (0, 0)),
            pl.BlockSpec((1, 32), lambda i: (0, 0)),
            pl.BlockSpec((32, 32), lambda i: (0, 0)),
            pl.BlockSpec((4, 32), lambda i: (0, 0)),
            pl.BlockSpec((BLK, 1), lambda i: (i, 0)),
            pl.BlockSpec((32, 1), lambda i: (0, 0)),
            pl.BlockSpec((1, 1), lambda i: (0, 0)),
        ],
        out_specs=pl.BlockSpec((1, G), lambda i: (0, 0)),
        out_shape=jax.ShapeDtypeStruct((1, G), jnp.float32),
        scratch_shapes=[
            pltpu.VMEM((1, G), jnp.float32),
            pltpu.VMEM((1, G), jnp.float32),
        ],
    )(hl, hr, t, t, sed, sed, wself, b, a, b4, batch_col, wout, bout)


@jax.jit
def kernel(state, action, edge_index, edge_attr, batch,
           W_in, b_in, W_msg0, W_self0, b0, W_msg1, W_self1, b1, W_out, b_out):
    ea8 = jnp.concatenate(
        [edge_attr,
         jnp.ones((E, 1), jnp.float32),
         jnp.zeros((E, 3), jnp.float32)], axis=1)
    z8 = jnp.zeros((ZCH, 8), jnp.float32)
    z16 = jnp.zeros((ZCH, 16), jnp.float32)

    sed = _make_sedeg_kernel()(ea8, edge_index, z8)

    hl, hr = _readin(state, action, W_in[:96], W_in[96:], b_in.reshape(1, C))

    spmm = _make_spmm_kernel()
    t = spmm(hl, hr, edge_index, z16)
    hl, hr = _layer(hl, hr, t, sed,
                    W_self0, b0.reshape(1, C), W_msg0[:C], W_msg0[C:])

    t = spmm(hl, hr, edge_index, z16)
    out = _layer_readout(hl, hr, t, sed,
                         W_self1, b1.reshape(1, C), W_msg1[:C], W_msg1[C:],
                         batch.reshape(N, 1), W_out, b_out.reshape(1, 1))
    return out.reshape(G, 1)
